# baseline (speedup 1.0000x reference)
import jax
import jax.numpy as jnp
from jax.experimental import pallas as pl

_VARIANCE = 1e-11


def _ln_body(x_ref, pos_ref, gamma_ref, beta_ref, out_ref):
    D = pos_ref.shape[-1]
    B = x_ref.shape[-1] // D
    pe = pos_ref[...]
    g = gamma_ref[0][None, :]
    bt = beta_ref[0][None, :]
    for b in range(B):
        v = x_ref[:, b * D:(b + 1) * D] + pe
        u = jnp.mean(v, axis=-1, keepdims=True)
        q = jnp.mean(v * v, axis=-1, keepdims=True)
        inv = jax.lax.rsqrt(q - u * u + _VARIANCE)
        out_ref[:, b * D:(b + 1) * D] = (v * inv - u * inv) * g + bt


def kernel(x, pos_table, gamma, beta):
    S, B, D = x.shape
    BS = 256
    grid = (S // BS,)
    x2 = x.reshape(S, B * D)
    gamma2 = gamma.reshape(1, D)
    beta2 = beta.reshape(1, D)
    out = pl.pallas_call(
        _ln_body,
        grid=grid,
        in_specs=[
            pl.BlockSpec((BS, B * D), lambda i: (i, 0)),
            pl.BlockSpec((BS, D), lambda i: (i, 0)),
            pl.BlockSpec((1, D), lambda i: (0, 0)),
            pl.BlockSpec((1, D), lambda i: (0, 0)),
        ],
        out_specs=pl.BlockSpec((BS, B * D), lambda i: (i, 0)),
        out_shape=jax.ShapeDtypeStruct((S, B * D), x.dtype),
    )(x2, pos_table, gamma2, beta2)
    return out.reshape(S, B, D)


# speedup vs baseline: 3.3523x; 3.3523x over previous
import jax
import jax.numpy as jnp
from jax.experimental import pallas as pl

_VARIANCE = 1e-11


def _ln_body(x_ref, pos_ref, gamma_ref, beta_ref, out_ref):
    BS, B, D = x_ref.shape
    pe = pos_ref[...]
    g = gamma_ref[0][None, :]
    bt = beta_ref[0][None, :]
    for b in range(B):
        v = x_ref[:, b, :] + pe
        u = jnp.mean(v, axis=-1, keepdims=True)
        q = jnp.mean(v * v, axis=-1, keepdims=True)
        inv = jax.lax.rsqrt(q - u * u + _VARIANCE)
        out_ref[:, b, :] = (v * inv - u * inv) * g + bt


def kernel(x, pos_table, gamma, beta):
    S, B, D = x.shape
    BS = 256
    grid = (S // BS,)
    gamma2 = gamma.reshape(1, D)
    beta2 = beta.reshape(1, D)
    return pl.pallas_call(
        _ln_body,
        grid=grid,
        in_specs=[
            pl.BlockSpec((BS, B, D), lambda i: (i, 0, 0)),
            pl.BlockSpec((BS, D), lambda i: (i, 0)),
            pl.BlockSpec((1, D), lambda i: (0, 0)),
            pl.BlockSpec((1, D), lambda i: (0, 0)),
        ],
        out_specs=pl.BlockSpec((BS, B, D), lambda i: (i, 0, 0)),
        out_shape=jax.ShapeDtypeStruct((S, B, D), x.dtype),
    )(x, pos_table, gamma2, beta2)
